# baseline (speedup 1.0000x reference)
import jax
import jax.numpy as jnp
from jax import lax
from jax.experimental import pallas as pl
from jax.experimental.pallas import tpu as pltpu
from jax.experimental.pallas import tpu_sc as plsc

_N = 50000
_E = 800000
_H = 32
_L = 3
_GW = 128
_NG = _E // _GW
_NC = 2
_NS = 16
_NW = _NC * _NS
_TG = 200
_NGPAD = _TG * _NW
_NP = 51200
_BR = 3200
_GRID = _NP // _BR

_BKT = 1568
_CAPL = 112
_SEG = 16 * _CAPL
_RPTC = _NW * _SEG
_RTOT = _NW * _RPTC
_CH = 512
_CROW = 1600


def _mesh():
    return plsc.VectorSubcoreMesh(core_axis_name="c", subcore_axis_name="s")


_SCPARAMS = pltpu.CompilerParams(use_tc_tiling_on_sc=False,
                                 needs_layout_passes=False)


def _partition(idx):
    def body(idx_ref, recs_ref, ibuf, stage_v, stage_i, tbuf, cnt, sem):
        cid = lax.axis_index("c")
        sid = lax.axis_index("s")
        wid = cid * _NS + sid
        iota = lax.broadcasted_iota(jnp.int32, (16,), 0)
        zeros16 = jnp.zeros((16,), jnp.int32)
        trash16 = jnp.left_shift(_BKT + iota, 16)

        def fill(i, c):
            tbuf[pl.ds(i * 16, 16)] = trash16
            return c
        lax.fori_loop(0, _SEG // 16, fill, 0)
        for i in range(_NW):
            cnt[pl.ds(i * 16, 16)] = zeros16
        cps = []
        for b in range(_NW):
            cps.append(pltpu.async_copy(
                tbuf, recs_ref.at[pl.ds(b * _RPTC + wid * _SEG, _SEG)],
                sem))
        for cp in cps:
            cp.wait()

        def chunk(ci, c):
            base = wid * _TG + ci * 8
            pltpu.sync_copy(idx_ref.at[pl.ds(base, 8)], ibuf)
            for g in range(64):
                j, r = g // 8, g % 8
                srcv = ibuf[j, 0, pl.ds(r * 16, 16)]
                dstv = ibuf[j, 1, pl.ds(r * 16, 16)]
                b = ((dstv.astype(jnp.float32) + 0.5)
                     * jnp.float32(1.0 / _BKT)).astype(jnp.int32)
                dloc = dstv - b * _BKT
                rec = jnp.bitwise_or(srcv, jnp.left_shift(dloc, 16))
                slot = b * 16 + iota
                pos = plsc.load_gather(cnt, [slot])
                plsc.store_scatter(cnt, [slot], pos + 1)
                pos = jnp.minimum(pos, _CAPL - 1)
                gpos = (b * _RPTC + wid * _SEG + iota * _CAPL) + pos
                stage_v[j, pl.ds(r * 16, 16)] = rec
                stage_i[j, pl.ds(r * 16, 16)] = gpos
            scps = []
            for j in range(8):
                scps.append(pltpu.async_copy(
                    stage_v.at[j], recs_ref.at[stage_i.at[j]], sem))
            for cp in scps:
                cp.wait()
            return c

        lax.fori_loop(0, _TG // 8, chunk, 0)

    f = pl.kernel(
        body,
        out_type=jax.ShapeDtypeStruct((_RTOT,), jnp.int32),
        mesh=_mesh(),
        scratch_types=(
            pltpu.VMEM((8, 2, _GW), jnp.int32),
            pltpu.VMEM((8, _GW), jnp.int32),
            pltpu.VMEM((8, _GW), jnp.int32),
            pltpu.VMEM((_SEG,), jnp.int32),
            pltpu.VMEM((_NW * 16,), jnp.int32),
            pltpu.SemaphoreType.DMA,
        ),
        compiler_params=_SCPARAMS)
    return f(idx)


def _bucket_pass(h, recs, zacc, with_counts):
    nacc = 2 * _CROW if with_counts else _CROW
    out_type = [jax.ShapeDtypeStruct((_NP, _H), jnp.float32)]
    if with_counts:
        out_type.append(jax.ShapeDtypeStruct((_NP, _H), jnp.float32))
    ngr = _CH // _GW

    def body(*refs):
        if with_counts:
            (h_ref, recs_ref, z_ref, out_ref, cnt_ref,
             rbuf, sidx, vote, rows, acc, s0, s1, s2, s3) = refs
        else:
            (h_ref, recs_ref, z_ref, out_ref,
             rbuf, sidx, vote, rows, acc, s0, s1, s2, s3) = refs
        sems = (s0, s1, s2, s3)
        cid = lax.axis_index("c")
        sid = lax.axis_index("s")
        wid = cid * _NS + sid
        iota = lax.broadcasted_iota(jnp.int32, (16,), 0)
        ones16 = jnp.ones((16,), jnp.float32)
        col0 = jnp.zeros((16,), jnp.int32)
        ridx = [iota + r * 16 for r in range(8)]
        cols = [jnp.full((16,), c, jnp.int32) for c in range(_H)]

        pltpu.sync_copy(z_ref.at[pl.ds(0, nacc)], acc)
        base = wid * _RPTC

        def chunk(ci, c):
            pltpu.sync_copy(recs_ref.at[pl.ds(base + ci * _CH, _CH)], rbuf)
            for g in range(ngr * 8):
                v = rbuf[pl.ds(g * 16, 16)]
                sidx[g // 8, pl.ds((g % 8) * 16, 16)] = \
                    jnp.bitwise_and(v, 0xFFFF)
            cps = [None] * 4
            cps[0] = pltpu.async_copy(h_ref.at[sidx.at[0]], rows.at[0],
                                      sems[0])
            for j in range(ngr):
                for a in range(1, 4):
                    if j + a < ngr and (j == 0 or a == 3):
                        nb = (j + a) % 4
                        cps[nb] = pltpu.async_copy(
                            h_ref.at[sidx.at[j + a]], rows.at[nb], sems[nb])
                cps[j % 4].wait()
                rv = rows.at[j % 4]
                for r in range(8):
                    v = rbuf[pl.ds((j * 8 + r) * 16, 16)]
                    dv = lax.shift_right_logical(v, 16)

                    def addgroup(win):
                        for c2 in range(_H):
                            vals = plsc.load_gather(
                                rv, [ridx[r], cols[c2]])
                            plsc.addupdate_scatter(acc, [dv, cols[c2]],
                                                   vals, mask=win)
                        if with_counts:
                            plsc.addupdate_scatter(
                                acc, [dv + _CROW, col0], ones16, mask=win)

                    plsc.store_scatter(vote, [dv], iota)
                    win = plsc.load_gather(vote, [dv]) == iota
                    addgroup(win)
                    loser0 = jnp.where(win, 0, 1)

                    def retry(maskv):
                        m = maskv > 0
                        plsc.store_scatter(vote, [dv], iota, mask=m)
                        w2 = m & (plsc.load_gather(vote, [dv]) == iota)
                        addgroup(w2)
                        return jnp.where(w2, 0, maskv)

                    lax.while_loop(lambda mv: jnp.any(mv > 0), retry,
                                   loser0)
            return c

        lax.fori_loop(0, _RPTC // _CH, chunk, 0)
        pltpu.sync_copy(acc.at[pl.ds(0, _BKT)],
                        out_ref.at[pl.ds(wid * _BKT, _BKT)])
        if with_counts:
            pltpu.sync_copy(acc.at[pl.ds(_CROW, _BKT)],
                            cnt_ref.at[pl.ds(wid * _BKT, _BKT)])

        @pl.when(wid == _NW - 1)
        def _tail():
            pad = _NP - _NW * _BKT
            pltpu.sync_copy(z_ref.at[pl.ds(0, pad)],
                            out_ref.at[pl.ds(_NW * _BKT, pad)])
            if with_counts:
                pltpu.sync_copy(z_ref.at[pl.ds(0, pad)],
                                cnt_ref.at[pl.ds(_NW * _BKT, pad)])

    f = pl.kernel(
        body,
        out_type=tuple(out_type),
        mesh=_mesh(),
        scratch_types=(
            pltpu.VMEM((_CH,), jnp.int32),
            pltpu.VMEM((4, _GW), jnp.int32),
            pltpu.VMEM((_CROW,), jnp.int32),
            pltpu.VMEM((4, _GW, _H), jnp.float32),
            pltpu.VMEM((nacc, _H), jnp.float32),
            pltpu.SemaphoreType.DMA,
            pltpu.SemaphoreType.DMA,
            pltpu.SemaphoreType.DMA,
            pltpu.SemaphoreType.DMA,
        ),
        compiler_params=_SCPARAMS)
    return f(h, recs, zacc)


def _tc_init(det, errf, wdet, bdet_, werr, berr_):
    def body(d_ref, e_ref, wd_ref, bd_ref, we_ref, be_ref, hd_ref, he_ref):
        hd_ref[...] = jnp.maximum(d_ref[...] * wd_ref[...] + bd_ref[...], 0.0)
        he_ref[...] = jnp.maximum(e_ref[...] * we_ref[...] + be_ref[...], 0.0)

    row = pl.BlockSpec((_BR, 1), lambda i: (i, 0))
    par = pl.BlockSpec((1, _H), lambda i: (0, 0))
    f = pl.pallas_call(
        body, grid=(_GRID,),
        in_specs=[row, row, par, par, par, par],
        out_specs=[pl.BlockSpec((_BR, _H), lambda i: (i, 0))] * 2,
        out_shape=[jax.ShapeDtypeStruct((_NP, _H), jnp.float32)] * 2,
    )
    return f(det, errf, wdet, bdet_, werr, berr_)


def _tc_update(h, s, c0, ew, wagg, wself, bias, g, b, weighted):
    def body(*refs):
        if weighted:
            (h_ref, s_ref, c_ref, ew_ref,
             wa_ref, ws_ref, bi_ref, g_ref, b_ref, o_ref) = refs
        else:
            (h_ref, s_ref, c_ref,
             wa_ref, ws_ref, bi_ref, g_ref, b_ref, o_ref) = refs
        agg = jnp.dot(s_ref[...], wa_ref[...],
                      preferred_element_type=jnp.float32)
        cnt = jnp.maximum(c_ref[...], 1.0)
        if weighted:
            scale = (1.0 / (1.0 + jnp.exp(-ew_ref[...]))) / cnt
        else:
            scale = 1.0 / cnt
        hcur = h_ref[...]
        pre = (jnp.dot(hcur, ws_ref[...], preferred_element_type=jnp.float32)
               + agg * scale + bi_ref[...])
        t = hcur + jnp.maximum(pre, 0.0)
        mu = jnp.mean(t, axis=1, keepdims=True)
        d = t - mu
        var = jnp.mean(d * d, axis=1, keepdims=True)
        o_ref[...] = d * lax.rsqrt(var + 1e-5) * g_ref[...] + b_ref[...]

    blk = pl.BlockSpec((_BR, _H), lambda i: (i, 0))
    col = pl.BlockSpec((_BR, 1), lambda i: (i, 0))
    wsp = pl.BlockSpec((_H, _H), lambda i: (0, 0))
    par = pl.BlockSpec((1, _H), lambda i: (0, 0))
    in_specs = [blk, blk, col]
    args = [h, s, c0]
    if weighted:
        in_specs.append(col)
        args.append(ew)
    in_specs += [wsp, wsp, par, par, par]
    args += [wagg, wself, bias, g, b]
    f = pl.pallas_call(
        body, grid=(_GRID,), in_specs=in_specs,
        out_specs=pl.BlockSpec((_BR, _H), lambda i: (i, 0)),
        out_shape=jax.ShapeDtypeStruct((_NP, _H), jnp.float32),
    )
    return f(*args)


def _tc_pool(he, maskf):
    def body(h_ref, m_ref, o_ref):
        i = pl.program_id(0)

        @pl.when(i == 0)
        def _init():
            o_ref[...] = jnp.zeros((8, _H), jnp.float32)
            o_ref[2:4, :] = jnp.full((2, _H), -jnp.inf, jnp.float32)

        h = h_ref[...]
        m = m_ref[...]
        rowid = (i * _BR
                 + lax.broadcasted_iota(jnp.int32, (_BR, 1), 0))
        valid = rowid < _N
        o_ref[0:1, :] += jnp.sum(jnp.where(m > 0.0, h, 0.0), axis=0,
                                 keepdims=True)
        o_ref[1:2, :] += jnp.sum(jnp.where(valid, h, 0.0), axis=0,
                                 keepdims=True)
        o_ref[2:3, :] = jnp.maximum(
            o_ref[2:3, :],
            jnp.max(jnp.where(m > 0.0, h, -jnp.inf), axis=0, keepdims=True))
        o_ref[3:4, :] = jnp.maximum(
            o_ref[3:4, :],
            jnp.max(jnp.where(valid, h, -jnp.inf), axis=0, keepdims=True))
        o_ref[4:5, :] += jnp.sum(m) * jnp.ones((1, _H), jnp.float32)

    f = pl.pallas_call(
        body, grid=(_GRID,),
        in_specs=[pl.BlockSpec((_BR, _H), lambda i: (i, 0)),
                  pl.BlockSpec((_BR, 1), lambda i: (i, 0))],
        out_specs=pl.BlockSpec((8, _H), lambda i: (0, 0)),
        out_shape=jax.ShapeDtypeStruct((8, _H), jnp.float32),
    )
    return f(he, maskf)


def _tc_head(stats, wh1, bh1_, wh2, bh2_):
    def body(s_ref, w1_ref, b1_ref, w2_ref, b2_ref, o_ref):
        s = s_ref[...]
        cnt = s[4:5, 0:1]
        use = cnt > 0.0
        mean_m = s[0:1, :] / jnp.maximum(cnt, 1.0)
        mean_p = s[1:2, :] * (1.0 / _N)
        mean_e = jnp.where(use, mean_m, mean_p)
        max_e = jnp.where(use, s[2:3, :], s[3:4, :])
        emb = jnp.concatenate([mean_e, max_e], axis=1)
        hmid = jnp.maximum(
            jnp.dot(emb, w1_ref[...], preferred_element_type=jnp.float32)
            + b1_ref[...], 0.0)
        o_ref[...] = (jnp.dot(hmid, w2_ref[...],
                              preferred_element_type=jnp.float32)
                      + b2_ref[...])

    f = pl.pallas_call(
        body, out_shape=jax.ShapeDtypeStruct((1, 1), jnp.float32))
    return f(stats, wh1, bh1_, wh2, bh2_)


def kernel(det_features, err_features, edge_index_d2e, edge_index_e2d,
           error_weights, observable_mask, Wdet, bdet, Werr, berr, Wd2e,
           We_self, be, ln_e_g, ln_e_b, We2d, Wd_self, bd, ln_d_g, ln_d_b,
           Wh1, bh1, Wh2, bh2):
    f32 = jnp.float32

    def _padrows(v):
        return jnp.pad(v, ((0, _NP - _N), (0, 0)))

    det = _padrows(det_features.reshape(_N, 1).astype(f32))
    errf = _padrows(err_features.reshape(_N, 1).astype(f32))

    def _prep_idx(ei):
        src = ei[0].reshape(_NG, _GW)
        dst = ei[1].reshape(_NG, _GW)
        both = jnp.stack([src, dst], axis=1)
        return jnp.pad(both, ((0, _NGPAD - _NG), (0, 0), (0, 0)),
                       constant_values=_N)

    idx_d2e = _prep_idx(edge_index_d2e)
    idx_e2d = _prep_idx(edge_index_e2d)
    zacc = jnp.zeros((2 * _CROW, _H), f32)
    ew2 = _padrows(error_weights.reshape(_N, 1).astype(f32))
    maskf = _padrows(observable_mask.reshape(_N, 1).astype(f32))

    recs_d2e = _partition(idx_d2e)
    recs_e2d = _partition(idx_e2d)
    hD, hE = _tc_init(det, errf, Wdet.reshape(1, _H), bdet.reshape(1, _H),
                      Werr.reshape(1, _H), berr.reshape(1, _H))

    ce = cd = None
    for k in range(_L):
        if k == 0:
            S, ce2 = _bucket_pass(hD, recs_d2e, zacc, True)
            ce = ce2[:, 0:1]
        else:
            (S,) = _bucket_pass(hD, recs_d2e, zacc, False)
        hE = _tc_update(hE, S, ce, ew2, Wd2e[k], We_self[k],
                        be[k].reshape(1, _H), ln_e_g[k].reshape(1, _H),
                        ln_e_b[k].reshape(1, _H), True)
        if k == 0:
            T, cd2 = _bucket_pass(hE, recs_e2d, zacc, True)
            cd = cd2[:, 0:1]
        else:
            (T,) = _bucket_pass(hE, recs_e2d, zacc, False)
        hD = _tc_update(hD, T, cd, None, We2d[k], Wd_self[k],
                        bd[k].reshape(1, _H), ln_d_g[k].reshape(1, _H),
                        ln_d_b[k].reshape(1, _H), False)

    stats = _tc_pool(hE, maskf)
    return _tc_head(stats, Wh1, bh1.reshape(1, _H), Wh2, bh2.reshape(1, 1))
